# baseline (speedup 1.0000x reference)
import functools

import jax
import jax.numpy as jnp
from jax import lax
from jax.experimental import pallas as pl
from jax.experimental.pallas import tpu as pltpu
from jax.experimental.pallas import tpu_sc as plsc

N_STATES = 1000000
N_SAMPLES = 16384
N_OBS = 16
LANES = 16
STRIDE = 16
N_BLOCKS = N_STATES // STRIDE
TBL = 1 << 16
NW = 32
PER_W = N_SAMPLES // NW
CHUNK = 128
N_CHUNKS = PER_W // CHUNK
N_GROUPS = CHUNK // LANES


def _rank16(tbl_ref, uu, nbits):
    c = jnp.zeros((LANES,), jnp.int32)
    for bit in range(nbits - 1, -1, -1):
        t = c + (1 << bit)
        vals = plsc.load_gather(tbl_ref, [t - 1])
        c = jnp.where(vals <= uu, t, c)
    return c


def _sc_sample_reduce(coarse, cdf2d, u, obs_evs):
    mesh = plsc.VectorSubcoreMesh(core_axis_name="c", subcore_axis_name="s")

    @functools.partial(
        pl.kernel,
        out_type=jax.ShapeDtypeStruct((NW, N_OBS), jnp.float32),
        mesh=mesh,
        compiler_params=pltpu.CompilerParams(
            needs_layout_passes=False, use_tc_tiling_on_sc=False),
        scratch_types=[
            pltpu.VMEM((TBL,), jnp.float32),
            pltpu.VMEM((PER_W,), jnp.float32),
            pltpu.VMEM((CHUNK,), jnp.int32),
            pltpu.VMEM((CHUNK, STRIDE), jnp.float32),
            pltpu.VMEM((CHUNK,), jnp.int32),
            pltpu.VMEM((CHUNK, N_OBS), jnp.float32),
            pltpu.VMEM((N_OBS,), jnp.float32),
            pltpu.SemaphoreType.DMA,
        ],
    )
    def k(coarse_hbm, cdf_hbm, u_hbm, obs_hbm, out_hbm,
          tbl_v, u_v, blk_v, rows_v, samp_v, obsr_v, acc_v, sem):
        wid = lax.axis_index("s") * 2 + lax.axis_index("c")
        pltpu.sync_copy(coarse_hbm, tbl_v)
        pltpu.sync_copy(u_hbm.at[pl.ds(wid * PER_W, PER_W)], u_v)
        acc = jnp.zeros((N_OBS,), jnp.float32)
        for ch in range(N_CHUNKS):
            for g in range(N_GROUPS):
                uu = u_v[pl.ds(ch * CHUNK + g * LANES, LANES)]
                c = _rank16(tbl_v, uu, 16)
                blk_v[pl.ds(g * LANES, LANES)] = jnp.minimum(c, N_BLOCKS - 1)
            pltpu.async_copy(cdf_hbm.at[blk_v], rows_v, sem).wait()
            for g in range(N_GROUPS):
                uu = u_v[pl.ds(ch * CHUNK + g * LANES, LANES)]
                cg = blk_v[pl.ds(g * LANES, LANES)]
                rows = lax.iota(jnp.int32, LANES) + g * LANES
                d = jnp.zeros((LANES,), jnp.int32)
                for bit in range(3, -1, -1):
                    t = d + (1 << bit)
                    vals = plsc.load_gather(rows_v, [rows, t - 1])
                    d = jnp.where(vals <= uu, t, d)
                samp_v[pl.ds(g * LANES, LANES)] = (cg << 4) + d
            pltpu.async_copy(obs_hbm.at[samp_v], obsr_v, sem).wait()

            def acc_body(i, a):
                return a + obsr_v[i]

            acc = lax.fori_loop(0, CHUNK, acc_body, acc)
        acc_v[...] = acc
        pltpu.sync_copy(acc_v, out_hbm.at[wid])

    return k(coarse, cdf2d, u, obs_evs)


def kernel(energies, u, obs_evs):
    logits = -energies
    logZ = jax.scipy.special.logsumexp(logits)
    probs = jnp.exp(logits - logZ)
    cdf = jnp.cumsum(probs)
    cdf2d = cdf.reshape(N_BLOCKS, STRIDE)
    coarse = jnp.concatenate(
        [cdf2d[:, STRIDE - 1],
         jnp.full((TBL - N_BLOCKS,), 2.0, jnp.float32)])
    partials = _sc_sample_reduce(coarse, cdf2d, u, obs_evs)
    return jnp.sum(partials, axis=0) / N_SAMPLES
